# Initial kernel scaffold; baseline (speedup 1.0000x reference)
import functools

import jax
import jax.numpy as jnp
import numpy as np
from jax import lax
from jax.experimental import pallas as pl
from jax.experimental.pallas import tpu as pltpu
from jax.experimental.pallas import tpu_sc as plsc

N = 10000
E = 320000
D = 128
H = 4
DH = D // H

NC = 2
NS = 16
NW = NC * NS
EPW = E // NW
EB = 80
NBLK = EPW // EB
RPT = N // NS
RCH = 125
NCH = RPT // RCH


def _proj_body(x_ref, w_ref, b_ref, q_ref, kv_ref):
    res = jnp.dot(x_ref[...], w_ref[...], preferred_element_type=jnp.float32)
    res = res + b_ref[...]
    q_ref[...] = res[:, :D]
    kv_ref[...] = res[:, D:]


def _proj(x, w1t, b1):
    bn = 1000
    return pl.pallas_call(
        _proj_body,
        grid=(N // bn,),
        in_specs=[
            pl.BlockSpec((bn, D), lambda i: (i, 0)),
            pl.BlockSpec((D, 3 * D), lambda i: (0, 0)),
            pl.BlockSpec((1, 3 * D), lambda i: (0, 0)),
        ],
        out_specs=[
            pl.BlockSpec((bn, D), lambda i: (i, 0)),
            pl.BlockSpec((bn, 2 * D), lambda i: (i, 0)),
        ],
        out_shape=[
            jax.ShapeDtypeStruct((N, D), jnp.float32),
            jax.ShapeDtypeStruct((N, 2 * D), jnp.float32),
        ],
    )(x, w1t, b1)


_mesh = plsc.VectorSubcoreMesh(
    core_axis_name="c", subcore_axis_name="s", num_cores=NC, num_subcores=NS
)


@functools.partial(
    pl.kernel,
    out_type=(
        jax.ShapeDtypeStruct((NC, N, D), jnp.float32),
        jax.ShapeDtypeStruct((NC, N, 16), jnp.float32),
    ),
    mesh=_mesh,
    scratch_types=[
        pltpu.VMEM((EB, 2 * D), jnp.float32),
        pltpu.VMEM((EB, D), jnp.float32),
        pltpu.VMEM((EB, D), jnp.float32),
        pltpu.VMEM((EB, 16), jnp.float32),
        pltpu.VMEM((EB,), jnp.int32),
        pltpu.VMEM((EB,), jnp.int32),
        pltpu.VMEM((RCH, D), jnp.float32),
        pltpu.VMEM((RCH, 16), jnp.float32),
        pltpu.VMEM_SHARED((N, D), jnp.float32),
        pltpu.VMEM_SHARED((N, 16), jnp.float32),
        pltpu.SemaphoreType.DMA,
    ],
)
def _edge_kernel(kv_hbm, q_hbm, src_hbm, dst_hbm, av_hbm, ad_hbm,
                 kv_buf, q_buf, w_buf, d_buf, src_buf, dst_buf,
                 zv, zd, acc_v, acc_d, sem):
    cid = lax.axis_index("c")
    sid = lax.axis_index("s")
    wid = cid * NS + sid

    zero16 = jnp.zeros((16,), jnp.float32)

    def _zero_row(r, _):
        for c in range(D // 16):
            zv[r, pl.ds(c * 16, 16)] = zero16
        zd[r] = zero16
        return 0

    lax.fori_loop(0, RCH, _zero_row, 0)

    def _zero_dbuf(r, _):
        d_buf[r] = zero16
        return 0

    lax.fori_loop(0, EB, _zero_dbuf, 0)

    for j in range(NCH):
        base = sid * RPT + j * RCH
        pltpu.sync_copy(zv, acc_v.at[pl.ds(base, RCH)])
        pltpu.sync_copy(zd, acc_d.at[pl.ds(base, RCH)])
    plsc.subcore_barrier()

    def _block(i, _):
        ebase = wid * EPW + i * EB
        pltpu.sync_copy(src_hbm.at[pl.ds(ebase, EB)], src_buf)
        pltpu.sync_copy(dst_hbm.at[pl.ds(ebase, EB)], dst_buf)
        pltpu.async_copy(kv_hbm.at[src_buf], kv_buf, sem).wait()
        pltpu.async_copy(q_hbm.at[dst_buf], q_buf, sem).wait()

        def _group(g, _):
            rows = lax.iota(jnp.int32, 16) + g * 16
            for h in range(H):
                acc = jnp.zeros((16,), jnp.float32)
                for d in range(h * DH, (h + 1) * DH):
                    dcol = jnp.full((16,), d, jnp.int32)
                    kc = plsc.load_gather(kv_buf, [rows, dcol])
                    qc = plsc.load_gather(q_buf, [rows, dcol])
                    acc = acc + kc * qc
                p = jnp.exp(acc)
                plsc.store_scatter(
                    d_buf, [rows, jnp.full((16,), h, jnp.int32)], p)
                for d in range(h * DH, (h + 1) * DH):
                    vc = plsc.load_gather(
                        kv_buf, [rows, jnp.full((16,), D + d, jnp.int32)])
                    plsc.store_scatter(
                        w_buf, [rows, jnp.full((16,), d, jnp.int32)], vc * p)
            return 0

        lax.fori_loop(0, EB // 16, _group, 0)
        pltpu.sync_copy(w_buf, acc_v.at[dst_buf], add=True)
        pltpu.sync_copy(d_buf, acc_d.at[dst_buf], add=True)
        return 0

    lax.fori_loop(0, NBLK, _block, 0)
    plsc.subcore_barrier()

    for j in range(NCH):
        base = sid * RPT + j * RCH
        pltpu.sync_copy(acc_v.at[pl.ds(base, RCH)], zv)
        pltpu.sync_copy(zv, av_hbm.at[cid, pl.ds(base, RCH)])
        pltpu.sync_copy(acc_d.at[pl.ds(base, RCH)], zd)
        pltpu.sync_copy(zd, ad_hbm.at[cid, pl.ds(base, RCH)])


def _out_body(av_ref, ad_ref, s_ref, w_ref, b_ref, o_ref):
    sum_v = av_ref[0] + av_ref[1]
    sum_d = ad_ref[0] + ad_ref[1]
    drep = jnp.dot(sum_d, s_ref[...], preferred_element_type=jnp.float32)
    pos = drep > 0.0
    safe = jnp.where(pos, sum_v, 0.0) / jnp.where(pos, drep, 1.0)
    o_ref[...] = (
        jnp.dot(safe, w_ref[...], preferred_element_type=jnp.float32)
        + b_ref[...]
    )


def _out_proj(av, ad, s_mat, wout_t, bout2d):
    bn = 1000
    return pl.pallas_call(
        _out_body,
        grid=(N // bn,),
        in_specs=[
            pl.BlockSpec((NC, bn, D), lambda i: (0, i, 0)),
            pl.BlockSpec((NC, bn, 16), lambda i: (0, i, 0)),
            pl.BlockSpec((16, D), lambda i: (0, 0)),
            pl.BlockSpec((D, D), lambda i: (0, 0)),
            pl.BlockSpec((1, D), lambda i: (0, 0)),
        ],
        out_specs=pl.BlockSpec((bn, D), lambda i: (i, 0)),
        out_shape=jax.ShapeDtypeStruct((N, D), jnp.float32),
    )(av, ad, s_mat, wout_t, bout2d)


_QPERM = np.concatenate([np.arange(h * 3 * DH, h * 3 * DH + DH)
                         for h in range(H)])
_KPERM = _QPERM + DH
_VPERM = _QPERM + 2 * DH

_SMAT = np.zeros((16, D), np.float32)
for _h in range(H):
    _SMAT[_h, _h * DH:(_h + 1) * DH] = 1.0
_SMAT = jnp.asarray(_SMAT)


def kernel(x, edge_index, Wqkv, bqkv, Wout, bout):
    coef = np.float32(1.0 / np.sqrt(DH))
    w1 = jnp.concatenate(
        [Wqkv[_QPERM], coef * Wqkv[_KPERM], Wqkv[_VPERM]], axis=0)
    b1 = jnp.concatenate(
        [bqkv[_QPERM], coef * bqkv[_KPERM], bqkv[_VPERM]])
    q, kv = _proj(x, w1.T, b1[None, :])
    src = edge_index[0]
    dst = edge_index[1]
    av, ad = _edge_kernel(kv, q, src, dst)
    return _out_proj(av, ad, _SMAT, Wout.T, bout[None, :])


# speedup vs baseline: 52.8028x; 52.8028x over previous
import functools

import jax
import jax.numpy as jnp
import numpy as np
from jax import lax
from jax.experimental import pallas as pl
from jax.experimental.pallas import tpu as pltpu
from jax.experimental.pallas import tpu_sc as plsc

N = 10000
E = 320000
D = 128
H = 4
DH = D // H

NC = 2
NS = 16
NW = NC * NS
EPW = E // NW
EB = 80
NBLK = EPW // EB
NP8 = 10240
RPT = NP8 // NS
RCH = 16
NCH = RPT // RCH
DR = NP8 // 32
DNT = 8
DRT = DR // DNT
DZC = 8


def _proj_body(x_ref, w_ref, b_ref, q_ref, kv_ref):
    res = jnp.dot(x_ref[...], w_ref[...], preferred_element_type=jnp.float32)
    res = res + b_ref[...]
    q_ref[...] = res[:, :D]
    kv_ref[...] = res[:, D:]


def _proj(x, w1t, b1):
    bn = 1000
    return pl.pallas_call(
        _proj_body,
        grid=(N // bn,),
        in_specs=[
            pl.BlockSpec((bn, D), lambda i: (i, 0)),
            pl.BlockSpec((D, 3 * D), lambda i: (0, 0)),
            pl.BlockSpec((1, 3 * D), lambda i: (0, 0)),
        ],
        out_specs=[
            pl.BlockSpec((bn, D), lambda i: (i, 0)),
            pl.BlockSpec((bn, 2 * D), lambda i: (i, 0)),
        ],
        out_shape=[
            jax.ShapeDtypeStruct((N, D), jnp.float32),
            jax.ShapeDtypeStruct((N, 2 * D), jnp.float32),
        ],
    )(x, w1t, b1)


@functools.cache
def _edge_kernel():
    mesh = plsc.VectorSubcoreMesh(
        core_axis_name="c", subcore_axis_name="s",
        num_cores=NC, num_subcores=NS,
    )
    return pl.kernel(
        _edge_body,
        out_type=jax.ShapeDtypeStruct((NC, NP8 + DR, D), jnp.float32),
        mesh=mesh,
        scratch_types=[
            pltpu.VMEM((EB, 2 * D), jnp.float32),
            pltpu.VMEM((EB, D), jnp.float32),
            pltpu.VMEM((EB, D), jnp.float32),
            pltpu.VMEM((EB,), jnp.int32),
            pltpu.VMEM((EB,), jnp.int32),
            pltpu.VMEM((EB,), jnp.int32),
            pltpu.VMEM((17 * 16 * 4,), jnp.float32),
            pltpu.VMEM((RCH, D), jnp.float32),
            pltpu.VMEM((DZC, D), jnp.float32),
            pltpu.VMEM_SHARED((NP8, D), jnp.float32),
            pltpu.VMEM_SHARED((DR, D), jnp.float32),
            pltpu.SemaphoreType.DMA,
            pltpu.SemaphoreType.DMA,
        ],
        compiler_params=pltpu.CompilerParams(needs_layout_passes=False),
    )


def _edge_body(kv_hbm, q_hbm, src_hbm, dst_hbm, out_hbm,
               kv_buf, q_buf, d_buf, src_buf, dst_buf, dstq_buf,
               flat_buf, zv, zd, acc_v, acc_d, sem, sem2):
    cid = lax.axis_index("c")
    sid = lax.axis_index("s")
    wid = cid * NS + sid

    zero16 = jnp.zeros((16,), jnp.float32)

    def _zero_row(r, _):
        for c in range(D // 16):
            zv[r, pl.ds(c * 16, 16)] = zero16
        return 0

    lax.fori_loop(0, RCH, _zero_row, 0)

    def _zero_zd(r, _):
        for c in range(D // 16):
            zd[r, pl.ds(c * 16, 16)] = zero16
        return 0

    lax.fori_loop(0, DZC, _zero_zd, 0)

    def _zero_dbuf(r, _):
        for c in range(D // 16):
            d_buf[r, pl.ds(c * 16, 16)] = zero16
        return 0

    lax.fori_loop(0, EB, _zero_dbuf, 0)

    for j in range(NCH):
        base = sid * RPT + j * RCH
        pltpu.sync_copy(zv, acc_v.at[pl.ds(base, RCH)])
    @pl.when(sid < DNT)
    def _zero_accd():
        for j in range(DRT // DZC):
            pltpu.sync_copy(zd, acc_d.at[pl.ds(sid * DRT + j * DZC, DZC)])

    plsc.subcore_barrier()

    def _block(i, _):
        ebase = wid * EPW + i * EB
        ci = pltpu.async_copy(src_hbm.at[pl.ds(ebase, EB)], src_buf, sem)
        ci2 = pltpu.async_copy(dst_hbm.at[pl.ds(ebase, EB)], dst_buf, sem2)
        ci.wait()
        ci2.wait()
        cg = pltpu.async_copy(kv_hbm.at[src_buf], kv_buf, sem)
        cg2 = pltpu.async_copy(q_hbm.at[dst_buf], q_buf, sem2)
        cg.wait()
        cg2.wait()

        lanes = lax.iota(jnp.int32, 16)

        def _group(g, _):
            rows = lanes + g * 16
            dstv = plsc.load_gather(dst_buf, [rows])
            plsc.store_scatter(dstq_buf, [rows],
                               lax.shift_right_logical(dstv, 5))
            dcol0 = lax.shift_left(lax.bitwise_and(dstv, 31), 2)

            @plsc.parallel_loop(0, 16, 1, unroll=8)
            def _p1(el):
                e = g * 16 + el
                for h in range(H):
                    c = h * DH
                    m = (kv_buf[e, pl.ds(c, 16)] * q_buf[e, pl.ds(c, 16)]
                         + kv_buf[e, pl.ds(c + 16, 16)]
                         * q_buf[e, pl.ds(c + 16, 16)])
                    plsc.store_scatter(
                        flat_buf, [lanes + (16 * h + el) * 17], m)

            P = []
            for h in range(H):
                s = jnp.zeros((16,), jnp.float32)
                for c in range(16):
                    s = s + plsc.load_gather(
                        flat_buf, [lanes * 17 + (17 * 16 * h + c)])
                P.append(jnp.exp(s))
                plsc.store_scatter(d_buf, [rows, dcol0 + h], P[h])

            @plsc.parallel_loop(0, 16, 1, unroll=8)
            def _p3(el):
                e = g * 16 + el
                idx = jnp.full((16,), el, jnp.int32)
                for h in range(H):
                    pb = jnp.take_along_axis(P[h], idx, axis=0)
                    c = h * DH
                    q_buf[e, pl.ds(c, 16)] = (
                        kv_buf[e, pl.ds(D + c, 16)] * pb)
                    q_buf[e, pl.ds(c + 16, 16)] = (
                        kv_buf[e, pl.ds(D + c + 16, 16)] * pb)

            return 0

        lax.fori_loop(0, EB // 16, _group, 0)
        cs = pltpu.async_copy(q_buf, acc_v.at[dst_buf], sem, add=True)
        cs2 = pltpu.async_copy(d_buf, acc_d.at[dstq_buf], sem2, add=True)
        cs.wait()
        cs2.wait()

        def _clean(g, _):
            rows = lax.iota(jnp.int32, 16) + g * 16
            dstv = plsc.load_gather(dst_buf, [rows])
            dcol0 = lax.shift_left(lax.bitwise_and(dstv, 31), 2)
            for h in range(H):
                plsc.store_scatter(d_buf, [rows, dcol0 + h], zero16)
            return 0

        lax.fori_loop(0, EB // 16, _clean, 0)
        return 0

    lax.fori_loop(0, NBLK, _block, 0)
    plsc.subcore_barrier()

    for j in range(NCH):
        base = sid * RPT + j * RCH
        pltpu.sync_copy(acc_v.at[pl.ds(base, RCH)], zv)
        pltpu.sync_copy(zv, out_hbm.at[cid, pl.ds(base, RCH)])
    @pl.when(sid < DNT)
    def _copyout_accd():
        for j in range(DRT // DZC):
            base = sid * DRT + j * DZC
            pltpu.sync_copy(acc_d.at[pl.ds(base, DZC)], zd)
            pltpu.sync_copy(zd, out_hbm.at[cid, pl.ds(NP8 + base, DZC)])


def _out_body(av_ref, ad_ref, s_ref, w_ref, b_ref, o_ref):
    sum_v = av_ref[0] + av_ref[1]
    sum_d = ad_ref[0] + ad_ref[1]
    drep = jnp.dot(sum_d, s_ref[...], preferred_element_type=jnp.float32)
    pos = drep > 0.0
    safe = jnp.where(pos, sum_v, 0.0) / jnp.where(pos, drep, 1.0)
    o_ref[...] = (
        jnp.dot(safe, w_ref[...], preferred_element_type=jnp.float32)
        + b_ref[...]
    )


def _out_proj(av, ad, s_mat, wout_t, bout2d):
    bn = 1000
    return pl.pallas_call(
        _out_body,
        grid=(N // bn,),
        in_specs=[
            pl.BlockSpec((NC, bn, D), lambda i: (0, i, 0)),
            pl.BlockSpec((NC, bn, H), lambda i: (0, i, 0)),
            pl.BlockSpec((H, D), lambda i: (0, 0)),
            pl.BlockSpec((D, D), lambda i: (0, 0)),
            pl.BlockSpec((1, D), lambda i: (0, 0)),
        ],
        out_specs=pl.BlockSpec((bn, D), lambda i: (i, 0)),
        out_shape=jax.ShapeDtypeStruct((N, D), jnp.float32),
    )(av, ad, s_mat, wout_t, bout2d)


_QPERM = np.concatenate([np.arange(h * 3 * DH, h * 3 * DH + DH)
                         for h in range(H)])
_KPERM = _QPERM + DH
_VPERM = _QPERM + 2 * DH

_SMAT = np.zeros((H, D), np.float32)
for _h in range(H):
    _SMAT[_h, _h * DH:(_h + 1) * DH] = 1.0


def kernel(x, edge_index, Wqkv, bqkv, Wout, bout):
    coef = np.float32(1.0 / np.sqrt(DH))
    w1 = jnp.concatenate(
        [Wqkv[_QPERM], coef * Wqkv[_KPERM], Wqkv[_VPERM]], axis=0)
    b1 = jnp.concatenate(
        [bqkv[_QPERM], coef * bqkv[_KPERM], bqkv[_VPERM]])
    q, kv = _proj(x, w1.T, b1[None, :])
    src = edge_index[0]
    dst = edge_index[1]
    fused = _edge_kernel()(kv, q, src, dst)
    av = fused[:, :N, :]
    ad = fused[:, NP8:, :].reshape(NC, NP8, H)[:, :N, :]
    return _out_proj(av, ad, jnp.asarray(_SMAT), Wout.T, bout[None, :])
